# Initial kernel scaffold; baseline (speedup 1.0000x reference)
import jax
import jax.numpy as jnp
from jax.experimental import pallas as pl


def kernel(repertoires):
    raise NotImplementedError("write your pallas kernel here")


# speedup vs baseline: 22.2797x; 22.2797x over previous
import functools

import jax
import jax.numpy as jnp
from jax import lax
from jax.experimental import pallas as pl
from jax.experimental.pallas import tpu as pltpu
from jax.experimental.pallas import tpu_sc as plsc

K = 3
A = 20
N_KMERS = A ** K
NBINS = 8192
LANES = 16

B, S, L = 4, 16384, 32
W = L - K + 1
NC, NS = 2, 16
NW = NC * NS
SEQ_PER_W = (B * S) // NW
WORDS_PER_W = SEQ_PER_W * L
GROUPS_PER_SC = NS // (NW // B)
ROWS_PER_GROUP = NW // B
COL_CHUNK = NBINS // ROWS_PER_GROUP


def _sc_kernel(rep_hbm, out_hbm, seq_buf, hist, red_buf, acc_buf, shared):
    c = lax.axis_index("c")
    s = lax.axis_index("s")
    group = s // ROWS_PER_GROUP
    slot = s % ROWS_PER_GROUP
    batch = NC * c + group
    row0 = batch * S + slot * SEQ_PER_W

    pltpu.sync_copy(rep_hbm.at[pl.ds(row0 * L, WORDS_PER_W + LANES)], seq_buf)

    def zero_body(i, _):
        hist[pl.ds(i * LANES, LANES)] = jnp.zeros((LANES,), jnp.float32)
        return 0
    lax.fori_loop(0, NBINS // LANES, zero_body, 0)

    ones = jnp.full((LANES,), 1.0, jnp.float32)
    lane = lax.broadcasted_iota(jnp.int32, (LANES,), 0)
    tail_mask = lane < (W - LANES)

    def seq_body(i, _):
        base = i * L
        a0 = seq_buf[pl.ds(base, LANES)]
        a1 = seq_buf[pl.ds(base + 1, LANES)]
        a2 = seq_buf[pl.ds(base + 2, LANES)]
        ids0 = a0 * (A * A) + a1 * A + a2
        b0 = seq_buf[pl.ds(base + LANES, LANES)]
        b1 = seq_buf[pl.ds(base + LANES + 1, LANES)]
        b2 = seq_buf[pl.ds(base + LANES + 2, LANES)]
        ids1 = b0 * (A * A) + b1 * A + b2
        plsc.addupdate_scatter(hist, [ids0], ones)
        plsc.addupdate_scatter(hist, [ids1], ones, mask=tail_mask)
        return 0
    lax.fori_loop(0, SEQ_PER_W, seq_body, 0)

    pltpu.sync_copy(hist, shared.at[s])
    plsc.subcore_barrier()

    rgroup = s // ROWS_PER_GROUP
    chunk = s % ROWS_PER_GROUP
    col0 = chunk * COL_CHUNK
    for r in range(ROWS_PER_GROUP):
        pltpu.sync_copy(
            shared.at[rgroup * ROWS_PER_GROUP + r, pl.ds(col0, COL_CHUNK)],
            red_buf.at[r])

    def red_body(j, _):
        acc = red_buf[0, pl.ds(j * LANES, LANES)]
        for r in range(1, ROWS_PER_GROUP):
            acc = acc + red_buf[r, pl.ds(j * LANES, LANES)]
        acc_buf[pl.ds(j * LANES, LANES)] = acc
        return 0
    lax.fori_loop(0, COL_CHUNK // LANES, red_body, 0)

    out_batch = NC * c + rgroup
    pltpu.sync_copy(acc_buf, out_hbm.at[out_batch, pl.ds(col0, COL_CHUNK)])


@jax.jit
def kernel(repertoires):
    flat = repertoires.reshape(-1).astype(jnp.int32)
    flat = jnp.concatenate([flat, jnp.zeros((LANES,), jnp.int32)])

    mesh = plsc.VectorSubcoreMesh(core_axis_name="c", subcore_axis_name="s")
    run = pl.kernel(
        _sc_kernel,
        mesh=mesh,
        compiler_params=pltpu.CompilerParams(needs_layout_passes=False),
        out_type=jax.ShapeDtypeStruct((B, NBINS), jnp.float32),
        scratch_types=[
            pltpu.VMEM((WORDS_PER_W + LANES,), jnp.int32),
            pltpu.VMEM((NBINS,), jnp.float32),
            pltpu.VMEM((ROWS_PER_GROUP, COL_CHUNK), jnp.float32),
            pltpu.VMEM((COL_CHUNK,), jnp.float32),
            pltpu.VMEM_SHARED((NS, NBINS), jnp.float32),
        ],
    )
    out = run(flat)
    return out[:, :N_KMERS]


# speedup vs baseline: 24.2671x; 1.0892x over previous
import jax
import jax.numpy as jnp
from jax import lax
from jax.experimental import pallas as pl
from jax.experimental.pallas import tpu as pltpu
from jax.experimental.pallas import tpu_sc as plsc

K = 3
A = 20
N_KMERS = A ** K
NBINS = 8192
LANES = 16

B, S, L = 4, 16384, 32
W = L - K + 1
NC, NS = 2, 16
NW = NC * NS
SEQ_PER_W = (B * S) // NW
ROWS_PER_GROUP = NW // B
COL_CHUNK = NBINS // ROWS_PER_GROUP


def _sc_kernel(rep_hbm, out_hbm, seq_buf, hist, red_buf, acc_buf, shared):
    c = lax.axis_index("c")
    s = lax.axis_index("s")
    batch = NC * c + s // ROWS_PER_GROUP
    row0 = batch * S + (s % ROWS_PER_GROUP) * SEQ_PER_W

    pltpu.sync_copy(rep_hbm.at[pl.ds(row0 * L, SEQ_PER_W * L)], seq_buf)

    def zero_body(i, _):
        hist[pl.ds(i * LANES, LANES)] = jnp.zeros((LANES,), jnp.float32)
        return 0
    lax.fori_loop(0, NBINS // LANES, zero_body, 0)

    ones = jnp.full((LANES,), 1.0, jnp.float32)
    lane = lax.broadcasted_iota(jnp.int32, (LANES,), 0)
    head_mask = lane >= 2

    def seq_body(i, _):
        base = i * L
        a0 = seq_buf[pl.ds(base, LANES)]
        a1 = seq_buf[pl.ds(base + 1, LANES)]
        a2 = seq_buf[pl.ds(base + 2, LANES)]
        ids0 = a0 * (A * A) + a1 * A + a2
        b0 = seq_buf[pl.ds(base + L - 2 - LANES, LANES)]
        b1 = seq_buf[pl.ds(base + L - 1 - LANES, LANES)]
        b2 = seq_buf[pl.ds(base + L - LANES, LANES)]
        ids1 = b0 * (A * A) + b1 * A + b2
        plsc.addupdate_scatter(hist, [ids0], ones)
        plsc.addupdate_scatter(hist, [ids1], ones, mask=head_mask)
        return 0
    lax.fori_loop(0, SEQ_PER_W, seq_body, 0)

    pltpu.sync_copy(hist, shared.at[s])
    plsc.subcore_barrier()

    rgroup = s // ROWS_PER_GROUP
    col0 = (s % ROWS_PER_GROUP) * COL_CHUNK
    for r in range(ROWS_PER_GROUP):
        pltpu.sync_copy(
            shared.at[rgroup * ROWS_PER_GROUP + r, pl.ds(col0, COL_CHUNK)],
            red_buf.at[r])

    def red_body(j, _):
        acc = red_buf[0, pl.ds(j * LANES, LANES)]
        for r in range(1, ROWS_PER_GROUP):
            acc = acc + red_buf[r, pl.ds(j * LANES, LANES)]
        acc_buf[pl.ds(j * LANES, LANES)] = acc
        return 0
    lax.fori_loop(0, COL_CHUNK // LANES, red_body, 0)

    out_batch = NC * c + rgroup
    pltpu.sync_copy(acc_buf, out_hbm.at[out_batch, pl.ds(col0, COL_CHUNK)])


@jax.jit
def kernel(repertoires):
    rep_flat = repertoires.reshape(-1)
    mesh = plsc.VectorSubcoreMesh(core_axis_name="c", subcore_axis_name="s")
    run = pl.kernel(
        _sc_kernel,
        mesh=mesh,
        compiler_params=pltpu.CompilerParams(needs_layout_passes=False),
        out_type=jax.ShapeDtypeStruct((B, NBINS), jnp.float32),
        scratch_types=[
            pltpu.VMEM((SEQ_PER_W * L,), jnp.int32),
            pltpu.VMEM((NBINS,), jnp.float32),
            pltpu.VMEM((ROWS_PER_GROUP, COL_CHUNK), jnp.float32),
            pltpu.VMEM((COL_CHUNK,), jnp.float32),
            pltpu.VMEM_SHARED((NS, NBINS), jnp.float32),
        ],
    )
    out = run(rep_flat)
    return out[:, :N_KMERS]


# speedup vs baseline: 28.5839x; 1.1779x over previous
import jax
import jax.numpy as jnp
from jax import lax
from jax.experimental import pallas as pl
from jax.experimental.pallas import tpu as pltpu
from jax.experimental.pallas import tpu_sc as plsc

K = 3
A = 20
N_KMERS = A ** K
NBINS = 8192
LANES = 16

B, S, L = 4, 16384, 32
W = L - K + 1
NC, NS = 2, 16
NW = NC * NS
SEQ_PER_W = (B * S) // NW
ROWS_PER_GROUP = NW // B
COL_CHUNK = NBINS // ROWS_PER_GROUP


def _sc_kernel(rep_hbm, out_hbm, seq_buf, hist, red_buf, acc_buf, shared, sem):
    c = lax.axis_index("c")
    s = lax.axis_index("s")
    batch = NC * c + s // ROWS_PER_GROUP
    row0 = batch * S + (s % ROWS_PER_GROUP) * SEQ_PER_W

    cp = pltpu.async_copy(rep_hbm.at[pl.ds(row0 * L, SEQ_PER_W * L)],
                          seq_buf, sem)

    @plsc.parallel_loop(0, NBINS, step=LANES, unroll=4)
    def zero_body(i):
        hist[pl.ds(i, LANES)] = jnp.zeros((LANES,), jnp.float32)
    cp.wait()

    ones = jnp.full((LANES,), 1.0, jnp.float32)
    lane = lax.broadcasted_iota(jnp.int32, (LANES,), 0)
    head_mask = lane >= 2

    @plsc.parallel_loop(0, SEQ_PER_W * L, step=L, unroll=4)
    def seq_body(base):
        a0 = seq_buf[pl.ds(base, LANES)]
        a1 = seq_buf[pl.ds(base + 1, LANES)]
        a2 = seq_buf[pl.ds(base + 2, LANES)]
        ids0 = a0 * (A * A) + a1 * A + a2
        b0 = seq_buf[pl.ds(base + L - 2 - LANES, LANES)]
        b1 = seq_buf[pl.ds(base + L - 1 - LANES, LANES)]
        b2 = seq_buf[pl.ds(base + L - LANES, LANES)]
        ids1 = b0 * (A * A) + b1 * A + b2
        plsc.addupdate_scatter(hist, [ids0], ones)
        plsc.addupdate_scatter(hist, [ids1], ones, mask=head_mask)

    pltpu.sync_copy(hist, shared.at[s])
    plsc.subcore_barrier()

    rgroup = s // ROWS_PER_GROUP
    col0 = (s % ROWS_PER_GROUP) * COL_CHUNK
    for r in range(ROWS_PER_GROUP):
        pltpu.sync_copy(
            shared.at[rgroup * ROWS_PER_GROUP + r, pl.ds(col0, COL_CHUNK)],
            red_buf.at[r])

    @plsc.parallel_loop(0, COL_CHUNK, step=LANES, unroll=4)
    def red_body(j):
        acc = red_buf[0, pl.ds(j, LANES)]
        for r in range(1, ROWS_PER_GROUP):
            acc = acc + red_buf[r, pl.ds(j, LANES)]
        acc_buf[pl.ds(j, LANES)] = acc

    out_batch = NC * c + rgroup
    pltpu.sync_copy(acc_buf, out_hbm.at[out_batch, pl.ds(col0, COL_CHUNK)])


@jax.jit
def kernel(repertoires):
    rep_flat = repertoires.reshape(-1)
    mesh = plsc.VectorSubcoreMesh(core_axis_name="c", subcore_axis_name="s")
    run = pl.kernel(
        _sc_kernel,
        mesh=mesh,
        compiler_params=pltpu.CompilerParams(needs_layout_passes=False),
        out_type=jax.ShapeDtypeStruct((B, NBINS), jnp.float32),
        scratch_types=[
            pltpu.VMEM((SEQ_PER_W * L,), jnp.int32),
            pltpu.VMEM((NBINS,), jnp.float32),
            pltpu.VMEM((ROWS_PER_GROUP, COL_CHUNK), jnp.float32),
            pltpu.VMEM((COL_CHUNK,), jnp.float32),
            pltpu.VMEM_SHARED((NS, NBINS), jnp.float32),
            pltpu.SemaphoreType.DMA,
        ],
    )
    out = run(rep_flat)
    return out[:, :N_KMERS]


# speedup vs baseline: 28.7671x; 1.0064x over previous
import jax
import jax.numpy as jnp
from jax import lax
from jax.experimental import pallas as pl
from jax.experimental.pallas import tpu as pltpu
from jax.experimental.pallas import tpu_sc as plsc

K = 3
A = 20
N_KMERS = A ** K
NBINS = 8192
LANES = 16

B, S, L = 4, 16384, 32
W = L - K + 1
NC, NS = 2, 16
NW = NC * NS
SEQ_PER_W = (B * S) // NW
ROWS_PER_GROUP = NW // B
COL_CHUNK = NBINS // ROWS_PER_GROUP


def _sc_kernel(rep_hbm, out_hbm, seq_buf, hist, red_buf, acc_buf, shared, sem):
    c = lax.axis_index("c")
    s = lax.axis_index("s")
    batch = NC * c + s // ROWS_PER_GROUP
    row0 = batch * S + (s % ROWS_PER_GROUP) * SEQ_PER_W

    cp = pltpu.async_copy(rep_hbm.at[pl.ds(row0 * L, SEQ_PER_W * L)],
                          seq_buf, sem)

    @plsc.parallel_loop(0, NBINS, step=LANES, unroll=4)
    def zero_body(i):
        hist[pl.ds(i, LANES)] = jnp.zeros((LANES,), jnp.float32)
    cp.wait()

    ones = jnp.full((LANES,), 1.0, jnp.float32)
    lane = lax.broadcasted_iota(jnp.int32, (LANES,), 0)
    head_mask = lane >= 2

    @plsc.parallel_loop(0, SEQ_PER_W * L, step=L, unroll=4)
    def seq_body(base):
        a0 = seq_buf[pl.ds(base, LANES)]
        a1 = seq_buf[pl.ds(base + 1, LANES)]
        a2 = seq_buf[pl.ds(base + 2, LANES)]
        ids0 = a0 * (A * A) + a1 * A + a2
        b0 = seq_buf[pl.ds(base + L - 2 - LANES, LANES)]
        b1 = seq_buf[pl.ds(base + L - 1 - LANES, LANES)]
        b2 = seq_buf[pl.ds(base + L - LANES, LANES)]
        ids1 = b0 * (A * A) + b1 * A + b2
        plsc.addupdate_scatter(hist, [ids0], ones)
        plsc.addupdate_scatter(hist, [ids1], ones, mask=head_mask)

    pltpu.sync_copy(hist, shared.at[pl.ds(s * NBINS, NBINS)])
    plsc.subcore_barrier()

    rgroup = s // ROWS_PER_GROUP
    col0 = (s % ROWS_PER_GROUP) * COL_CHUNK
    for r in range(ROWS_PER_GROUP):
        pltpu.sync_copy(
            shared.at[pl.ds((rgroup * ROWS_PER_GROUP + r) * NBINS + col0,
                            COL_CHUNK)],
            red_buf.at[pl.ds(r * COL_CHUNK, COL_CHUNK)])

    @plsc.parallel_loop(0, COL_CHUNK, step=LANES, unroll=4)
    def red_body(j):
        acc = red_buf[pl.ds(j, LANES)]
        for r in range(1, ROWS_PER_GROUP):
            acc = acc + red_buf[pl.ds(r * COL_CHUNK + j, LANES)]
        acc_buf[pl.ds(j, LANES)] = acc

    out_batch = NC * c + rgroup
    pltpu.sync_copy(acc_buf,
                    out_hbm.at[pl.ds(out_batch * NBINS + col0, COL_CHUNK)])


@jax.jit
def kernel(repertoires):
    rep_flat = repertoires.reshape(-1)
    mesh = plsc.VectorSubcoreMesh(core_axis_name="c", subcore_axis_name="s")
    run = pl.kernel(
        _sc_kernel,
        mesh=mesh,
        compiler_params=pltpu.CompilerParams(needs_layout_passes=False,
                                             use_tc_tiling_on_sc=True),
        out_type=jax.ShapeDtypeStruct((B * NBINS,), jnp.float32),
        scratch_types=[
            pltpu.VMEM((SEQ_PER_W * L,), jnp.int32),
            pltpu.VMEM((NBINS,), jnp.float32),
            pltpu.VMEM((ROWS_PER_GROUP * COL_CHUNK,), jnp.float32),
            pltpu.VMEM((COL_CHUNK,), jnp.float32),
            pltpu.VMEM_SHARED((NS * NBINS,), jnp.float32),
            pltpu.SemaphoreType.DMA,
        ],
    )
    out = run(rep_flat)
    return out.reshape(B, NBINS)[:, :N_KMERS]
